# baseline (speedup 1.0000x reference)
import functools

import jax
import jax.numpy as jnp
from jax import lax
from jax.experimental import pallas as pl
from jax.experimental.pallas import tpu as pltpu
from jax.experimental.pallas import tpu_sc as plsc

N = 10000
E = 160000
D = 128
DP = 144
NT = 16
CHUNK = 64
NCHUNK = 160
EPTP = NCHUNK * CHUNK
RPT = N // NT
BM = 1000


def _mm_body(h_ref, w_ref, o_ref):
    acc = jnp.dot(h_ref[...], w_ref[0],
                  preferred_element_type=jnp.float32,
                  precision=lax.Precision.HIGHEST)
    col = lax.broadcasted_iota(jnp.int32, (BM, DP), 1)
    o_ref[...] = acc + jnp.where(col == D, 1.0, 0.0)


def _fin_body(a0_ref, a1_ref, o_ref):
    x0 = a0_ref[...]
    x1 = a1_ref[...]
    d0 = jnp.maximum(x0[:, D:D + 1], 1.0)
    d1 = jnp.maximum(x1[:, D:D + 1], 1.0)
    o_ref[...] = jnp.maximum(x0[:, :D] / d0 + x1[:, :D] / d1, 0.0)


@functools.partial(
    pl.kernel,
    out_type=jax.ShapeDtypeStruct((2 * N, DP), jnp.float32),
    mesh=plsc.VectorSubcoreMesh(core_axis_name="c", subcore_axis_name="s"),
    scratch_types=[
        pltpu.VMEM((CHUNK,), jnp.int32),
        pltpu.VMEM((CHUNK,), jnp.int32),
        pltpu.VMEM((CHUNK,), jnp.int32),
        pltpu.VMEM((CHUNK,), jnp.int32),
        pltpu.VMEM((CHUNK,), jnp.int32),
        pltpu.VMEM((CHUNK,), jnp.int32),
        pltpu.VMEM((CHUNK,), jnp.int32),
        pltpu.VMEM((CHUNK,), jnp.int32),
        pltpu.VMEM((4, CHUNK, DP), jnp.float32),
        pltpu.VMEM_SHARED((N + 16, DP), jnp.float32),
        pltpu.SemaphoreType.DMA,
        pltpu.SemaphoreType.DMA,
        pltpu.SemaphoreType.DMA,
        pltpu.SemaphoreType.DMA,
        pltpu.SemaphoreType.DMA,
        pltpu.SemaphoreType.DMA,
        pltpu.SemaphoreType.DMA,
        pltpu.SemaphoreType.DMA,
    ],
    compiler_params=pltpu.CompilerParams(use_tc_tiling_on_sc=False),
)
def _sc_aggregate(hwp_hbm, sd_hbm, out_hbm,
                  s0, s1, s2, s3, d0, d1, d2, d3,
                  rows_v, agg_s, g0, g1, g2, g3, i0, i1, i2, i3):
    c = lax.axis_index("c")
    s = lax.axis_index("s")
    gsem = (g0, g1, g2, g3)
    isem = (i0, i1, i2, i3)
    sref = (s0, s1, s2, s3)
    dref = (d0, d1, d2, d3)

    zrow = jnp.zeros((16,), jnp.float32)

    def zbody(i, carry):
        for k in range(DP // 16):
            rows_v[0, i, pl.ds(k * 16, 16)] = zrow
        return carry

    lax.fori_loop(0, CHUNK, zbody, 0)
    for j in range(RPT // CHUNK):
        pltpu.sync_copy(rows_v.at[0],
                        agg_s.at[pl.ds(s * RPT + j * CHUNK, CHUNK)])
    rem = RPT - (RPT // CHUNK) * CHUNK
    pltpu.sync_copy(rows_v.at[0, pl.ds(0, rem)],
                    agg_s.at[pl.ds(s * RPT + RPT - rem, rem)])
    plsc.subcore_barrier()

    def quad(k, carry):
        j = 4 * k
        icps = []
        for b in range(4):
            ia = pltpu.async_copy(sd_hbm.at[c, s, j + b, 0], sref[b], isem[b])
            ib = pltpu.async_copy(sd_hbm.at[c, s, j + b, 1], dref[b], isem[b])
            icps.append((ia, ib))
        gcps = []
        for b in range(4):
            icps[b][0].wait()
            icps[b][1].wait()
            gcps.append(
                pltpu.async_copy(hwp_hbm.at[sref[b]], rows_v.at[b], gsem[b]))
        for b in range(4):
            gcps[b].wait()
            pltpu.sync_copy(rows_v.at[b], agg_s.at[dref[b]], add=True)
        return carry

    lax.fori_loop(0, NCHUNK // 4, quad, 0)
    plsc.subcore_barrier()

    pltpu.sync_copy(agg_s.at[pl.ds(s * RPT, RPT)],
                    out_hbm.at[pl.ds(c * N + s * RPT, RPT)])


def kernel(inp_h, edge_index_e0, edge_index_e1, W_e0, W_e1):
    pad = EPTP - E // NT
    src = jnp.stack([edge_index_e0[0], edge_index_e1[0] + N])
    src = jnp.pad(src.reshape(2, NT, E // NT), ((0, 0), (0, 0), (0, pad)))
    dst = jnp.stack([edge_index_e0[1], edge_index_e1[1]])
    dst = jnp.pad(dst.reshape(2, NT, E // NT), ((0, 0), (0, 0), (0, pad)),
                  constant_values=N)
    sd = jnp.stack([src.reshape(2, NT, NCHUNK, CHUNK),
                    dst.reshape(2, NT, NCHUNK, CHUNK)],
                   axis=3)
    wp = jnp.zeros((2, D, DP), jnp.float32)
    wp = wp.at[:, :, :D].set(jnp.stack([W_e0, W_e1]))

    hwp = pl.pallas_call(
        _mm_body,
        grid=(2, N // BM),
        in_specs=[
            pl.BlockSpec((BM, D), lambda r, i: (i, 0)),
            pl.BlockSpec((1, D, DP), lambda r, i: (r, 0, 0)),
        ],
        out_specs=pl.BlockSpec((BM, DP), lambda r, i: (r * (N // BM) + i, 0)),
        out_shape=jax.ShapeDtypeStruct((2 * N, DP), jnp.float32),
    )(inp_h, wp)

    agg = _sc_aggregate(hwp, sd)

    out = pl.pallas_call(
        _fin_body,
        grid=(N // BM,),
        in_specs=[
            pl.BlockSpec((BM, DP), lambda i: (i, 0)),
            pl.BlockSpec((BM, DP), lambda i: (i + N // BM, 0)),
        ],
        out_specs=pl.BlockSpec((BM, D), lambda i: (i, 0)),
        out_shape=jax.ShapeDtypeStruct((N, D), jnp.float32),
    )(agg, agg)
    return out


# speedup vs baseline: 1.0256x; 1.0256x over previous
import functools

import jax
import jax.numpy as jnp
from jax import lax
from jax.experimental import pallas as pl
from jax.experimental.pallas import tpu as pltpu
from jax.experimental.pallas import tpu_sc as plsc

N = 10000
E = 160000
D = 128
DP = 144
NT = 16
CHUNK = 64
NCHUNK = 160
EPTP = NCHUNK * CHUNK
RPT = N // NT
BM = 1000


def _mm_body(h_ref, w_ref, o_ref):
    acc = jnp.dot(h_ref[...], w_ref[0],
                  preferred_element_type=jnp.float32,
                  precision=lax.Precision.HIGHEST)
    col = lax.broadcasted_iota(jnp.int32, (BM, DP), 1)
    o_ref[...] = acc + jnp.where(col == D, 1.0, 0.0)


def _fin_body(a0_ref, a1_ref, o_ref):
    x0 = a0_ref[...]
    x1 = a1_ref[...]
    d0 = jnp.maximum(x0[:, D:D + 1], 1.0)
    d1 = jnp.maximum(x1[:, D:D + 1], 1.0)
    o_ref[...] = jnp.maximum(x0[:, :D] / d0 + x1[:, :D] / d1, 0.0)


@functools.partial(
    pl.kernel,
    out_type=jax.ShapeDtypeStruct((2 * N, DP), jnp.float32),
    mesh=plsc.VectorSubcoreMesh(core_axis_name="c", subcore_axis_name="s"),
    scratch_types=(
        [pltpu.VMEM((CHUNK,), jnp.int32) for _ in range(8)]
        + [pltpu.VMEM((CHUNK,), jnp.int32) for _ in range(8)]
        + [
            pltpu.VMEM((4, CHUNK, DP), jnp.float32),
            pltpu.VMEM_SHARED((N + 16, DP), jnp.float32),
        ]
        + [pltpu.SemaphoreType.DMA for _ in range(4)]
        + [pltpu.SemaphoreType.DMA for _ in range(8)]
    ),
    compiler_params=pltpu.CompilerParams(use_tc_tiling_on_sc=False),
)
def _sc_aggregate(hwp_hbm, sd_hbm, out_hbm, *refs):
    sref = refs[0:8]
    dref = refs[8:16]
    rows_v = refs[16]
    agg_s = refs[17]
    gsem = refs[18:22]
    isem = refs[22:30]
    c = lax.axis_index("c")
    s = lax.axis_index("s")

    zrow = jnp.zeros((16,), jnp.float32)

    def zbody(i, carry):
        for k in range(DP // 16):
            rows_v[0, i, pl.ds(k * 16, 16)] = zrow
        return carry

    lax.fori_loop(0, CHUNK, zbody, 0)
    for j in range(RPT // CHUNK):
        pltpu.sync_copy(rows_v.at[0],
                        agg_s.at[pl.ds(s * RPT + j * CHUNK, CHUNK)])
    rem = RPT - (RPT // CHUNK) * CHUNK
    pltpu.sync_copy(rows_v.at[0, pl.ds(0, rem)],
                    agg_s.at[pl.ds(s * RPT + RPT - rem, rem)])
    plsc.subcore_barrier()

    def idx_start(j, slot):
        pltpu.async_copy(sd_hbm.at[c, s, j, 0], sref[slot], isem[slot])
        pltpu.async_copy(sd_hbm.at[c, s, j, 1], dref[slot], isem[slot])

    def idx_wait(slot):
        pltpu.make_async_copy(sd_hbm.at[c, s, 0, 0], sref[slot],
                              isem[slot]).wait()
        pltpu.make_async_copy(sd_hbm.at[c, s, 0, 1], dref[slot],
                              isem[slot]).wait()

    def phase(j, base):
        gcps = []
        for b in range(4):
            idx_wait(base + b)
            gcps.append(pltpu.async_copy(hwp_hbm.at[sref[base + b]],
                                         rows_v.at[b], gsem[b]))
        for b in range(4):
            gcps[b].wait()
            pltpu.sync_copy(rows_v.at[b], agg_s.at[dref[base + b]], add=True)
            idx_start(jnp.minimum(j + b + 8, NCHUNK - 1), base + b)

    for b in range(4):
        idx_start(b, b)
        idx_start(b + 4, b + 4)

    def body8(m, carry):
        j = 8 * m
        phase(j, 0)
        phase(j + 4, 4)
        return carry

    lax.fori_loop(0, NCHUNK // 8, body8, 0)
    for b in range(8):
        idx_wait(b)
    plsc.subcore_barrier()

    pltpu.sync_copy(agg_s.at[pl.ds(s * RPT, RPT)],
                    out_hbm.at[pl.ds(c * N + s * RPT, RPT)])


def kernel(inp_h, edge_index_e0, edge_index_e1, W_e0, W_e1):
    pad = EPTP - E // NT
    src = jnp.stack([edge_index_e0[0], edge_index_e1[0] + N])
    src = jnp.pad(src.reshape(2, NT, E // NT), ((0, 0), (0, 0), (0, pad)))
    dst = jnp.stack([edge_index_e0[1], edge_index_e1[1]])
    dst = jnp.pad(dst.reshape(2, NT, E // NT), ((0, 0), (0, 0), (0, pad)),
                  constant_values=N)
    sd = jnp.stack([src.reshape(2, NT, NCHUNK, CHUNK),
                    dst.reshape(2, NT, NCHUNK, CHUNK)],
                   axis=3)
    wp = jnp.zeros((2, D, DP), jnp.float32)
    wp = wp.at[:, :, :D].set(jnp.stack([W_e0, W_e1]))

    hwp = pl.pallas_call(
        _mm_body,
        grid=(2, N // BM),
        in_specs=[
            pl.BlockSpec((BM, D), lambda r, i: (i, 0)),
            pl.BlockSpec((1, D, DP), lambda r, i: (r, 0, 0)),
        ],
        out_specs=pl.BlockSpec((BM, DP), lambda r, i: (r * (N // BM) + i, 0)),
        out_shape=jax.ShapeDtypeStruct((2 * N, DP), jnp.float32),
    )(inp_h, wp)

    agg = _sc_aggregate(hwp, sd)

    out = pl.pallas_call(
        _fin_body,
        grid=(N // BM,),
        in_specs=[
            pl.BlockSpec((BM, DP), lambda i: (i, 0)),
            pl.BlockSpec((BM, DP), lambda i: (i + N // BM, 0)),
        ],
        out_specs=pl.BlockSpec((BM, D), lambda i: (i, 0)),
        out_shape=jax.ShapeDtypeStruct((N, D), jnp.float32),
    )(agg, agg)
    return out


# speedup vs baseline: 1.0371x; 1.0112x over previous
import functools

import jax
import jax.numpy as jnp
from jax import lax
from jax.experimental import pallas as pl
from jax.experimental.pallas import tpu as pltpu
from jax.experimental.pallas import tpu_sc as plsc

N = 10000
E = 160000
D = 128
DP = 144
NT = 16
CHUNK = 64
NCHUNK = 160
EPTP = NCHUNK * CHUNK
RPT = N // NT
BM = 1000


def _mm_body(h_ref, w_ref, o_ref):
    acc = jnp.dot(h_ref[...], w_ref[0],
                  preferred_element_type=jnp.float32,
                  precision=lax.Precision.HIGHEST)
    col = lax.broadcasted_iota(jnp.int32, (BM, DP), 1)
    o_ref[...] = acc + jnp.where(col == D, 1.0, 0.0)


def _fin_body(a0_ref, a1_ref, o_ref):
    x0 = a0_ref[...]
    x1 = a1_ref[...]
    d0 = jnp.maximum(x0[:, D:D + 1], 1.0)
    d1 = jnp.maximum(x1[:, D:D + 1], 1.0)
    o_ref[...] = jnp.maximum(x0[:, :D] / d0 + x1[:, :D] / d1, 0.0)


@functools.partial(
    pl.kernel,
    out_type=jax.ShapeDtypeStruct((2 * N, DP), jnp.float32),
    mesh=plsc.VectorSubcoreMesh(core_axis_name="c", subcore_axis_name="s"),
    scratch_types=(
        [pltpu.VMEM((CHUNK,), jnp.int32) for _ in range(8)]
        + [pltpu.VMEM((CHUNK,), jnp.int32) for _ in range(8)]
        + [
            pltpu.VMEM((4, CHUNK, DP), jnp.float32),
            pltpu.VMEM_SHARED((N + 16, DP), jnp.float32),
        ]
        + [pltpu.SemaphoreType.DMA for _ in range(4)]
        + [pltpu.SemaphoreType.DMA for _ in range(8)]
        + [pltpu.SemaphoreType.DMA for _ in range(4)]
    ),
    compiler_params=pltpu.CompilerParams(use_tc_tiling_on_sc=False),
)
def _sc_aggregate(hwp_hbm, sd_hbm, out_hbm, *refs):
    sref = refs[0:8]
    dref = refs[8:16]
    rows_v = refs[16]
    agg_s = refs[17]
    gsem = refs[18:22]
    isem = refs[22:30]
    ssem = refs[30:34]
    c = lax.axis_index("c")
    s = lax.axis_index("s")

    zrow = jnp.zeros((16,), jnp.float32)

    def zbody(i, carry):
        for k in range(DP // 16):
            rows_v[0, i, pl.ds(k * 16, 16)] = zrow
        return carry

    lax.fori_loop(0, CHUNK, zbody, 0)
    for j in range(RPT // CHUNK):
        pltpu.sync_copy(rows_v.at[0],
                        agg_s.at[pl.ds(s * RPT + j * CHUNK, CHUNK)])
    rem = RPT - (RPT // CHUNK) * CHUNK
    pltpu.sync_copy(rows_v.at[0, pl.ds(0, rem)],
                    agg_s.at[pl.ds(s * RPT + RPT - rem, rem)])
    plsc.subcore_barrier()

    def idx_start(j, slot):
        pltpu.async_copy(sd_hbm.at[c, s, j, 0], sref[slot], isem[slot])
        pltpu.async_copy(sd_hbm.at[c, s, j, 1], dref[slot], isem[slot])

    def idx_wait(slot):
        pltpu.make_async_copy(sd_hbm.at[c, s, 0, 0], sref[slot],
                              isem[slot]).wait()
        pltpu.make_async_copy(sd_hbm.at[c, s, 0, 1], dref[slot],
                              isem[slot]).wait()

    def phase(j, base):
        gcps = []
        for b in range(4):
            idx_wait(base + b)
            gcps.append(pltpu.async_copy(hwp_hbm.at[sref[base + b]],
                                         rows_v.at[b], gsem[b]))
        scps = []
        for b in range(4):
            gcps[b].wait()
            scps.append(pltpu.async_copy(rows_v.at[b],
                                         agg_s.at[dref[base + b]],
                                         ssem[b], add=True))
        for b in range(4):
            scps[b].wait()
            idx_start(jnp.minimum(j + b + 8, NCHUNK - 1), base + b)

    for b in range(4):
        idx_start(b, b)
        idx_start(b + 4, b + 4)

    def body8(m, carry):
        j = 8 * m
        phase(j, 0)
        phase(j + 4, 4)
        return carry

    lax.fori_loop(0, NCHUNK // 8, body8, 0)
    for b in range(8):
        idx_wait(b)
    plsc.subcore_barrier()

    pltpu.sync_copy(agg_s.at[pl.ds(s * RPT, RPT)],
                    out_hbm.at[pl.ds(c * N + s * RPT, RPT)])


def kernel(inp_h, edge_index_e0, edge_index_e1, W_e0, W_e1):
    pad = EPTP - E // NT
    src = jnp.stack([edge_index_e0[0], edge_index_e1[0] + N])
    src = jnp.pad(src.reshape(2, NT, E // NT), ((0, 0), (0, 0), (0, pad)))
    dst = jnp.stack([edge_index_e0[1], edge_index_e1[1]])
    dst = jnp.pad(dst.reshape(2, NT, E // NT), ((0, 0), (0, 0), (0, pad)),
                  constant_values=N)
    sd = jnp.stack([src.reshape(2, NT, NCHUNK, CHUNK),
                    dst.reshape(2, NT, NCHUNK, CHUNK)],
                   axis=3)
    wp = jnp.zeros((2, D, DP), jnp.float32)
    wp = wp.at[:, :, :D].set(jnp.stack([W_e0, W_e1]))

    hwp = pl.pallas_call(
        _mm_body,
        grid=(2, N // BM),
        in_specs=[
            pl.BlockSpec((BM, D), lambda r, i: (i, 0)),
            pl.BlockSpec((1, D, DP), lambda r, i: (r, 0, 0)),
        ],
        out_specs=pl.BlockSpec((BM, DP), lambda r, i: (r * (N // BM) + i, 0)),
        out_shape=jax.ShapeDtypeStruct((2 * N, DP), jnp.float32),
    )(inp_h, wp)

    agg = _sc_aggregate(hwp, sd)

    out = pl.pallas_call(
        _fin_body,
        grid=(N // BM,),
        in_specs=[
            pl.BlockSpec((BM, DP), lambda i: (i, 0)),
            pl.BlockSpec((BM, DP), lambda i: (i + N // BM, 0)),
        ],
        out_specs=pl.BlockSpec((BM, D), lambda i: (i, 0)),
        out_shape=jax.ShapeDtypeStruct((N, D), jnp.float32),
    )(agg, agg)
    return out


# speedup vs baseline: 2.2193x; 2.1399x over previous
import functools

import jax
import jax.numpy as jnp
from jax import lax
from jax.experimental import pallas as pl
from jax.experimental.pallas import tpu as pltpu
from jax.experimental.pallas import tpu_sc as plsc

N = 10000
E = 160000
D = 128
DP = 144
NT = 16
CHUNK = 125
G = 8
NGRP = (E // NT) // (G * CHUNK)
RPT = N // NT
BM = 1000


def _mm_body(h_ref, w_ref, o_ref):
    acc = jnp.dot(h_ref[...], w_ref[0],
                  preferred_element_type=jnp.float32,
                  precision=lax.Precision.HIGHEST)
    col = lax.broadcasted_iota(jnp.int32, (BM, DP), 1)
    o_ref[...] = acc + jnp.where(col == D, 1.0, 0.0)


def _fin_body(a0_ref, a1_ref, o_ref):
    x0 = a0_ref[...]
    x1 = a1_ref[...]
    d0 = jnp.maximum(x0[:, D:D + 1], 1.0)
    d1 = jnp.maximum(x1[:, D:D + 1], 1.0)
    o_ref[...] = jnp.maximum(x0[:, :D] / d0 + x1[:, :D] / d1, 0.0)


@functools.partial(
    pl.kernel,
    out_type=jax.ShapeDtypeStruct((2 * N, DP), jnp.float32),
    mesh=plsc.VectorSubcoreMesh(core_axis_name="c", subcore_axis_name="s"),
    scratch_types=[
        pltpu.VMEM((G, 2, CHUNK), jnp.int32),
        pltpu.VMEM((2, CHUNK, DP), jnp.float32),
        pltpu.VMEM_SHARED((N, DP), jnp.float32),
        pltpu.SemaphoreType.DMA,
        pltpu.SemaphoreType.DMA,
        pltpu.SemaphoreType.DMA,
    ],
    compiler_params=pltpu.CompilerParams(use_tc_tiling_on_sc=False),
)
def _sc_aggregate(hwp_hbm, sd_hbm, out_hbm,
                  igrp, rows_v, agg_s, gsem, ss0, ss1):
    c = lax.axis_index("c")
    s = lax.axis_index("s")
    ssem = (ss0, ss1)

    zrow = jnp.zeros((16,), jnp.float32)

    def zbody(i, carry):
        for k in range(DP // 16):
            rows_v[0, i, pl.ds(k * 16, 16)] = zrow
        return carry

    lax.fori_loop(0, CHUNK, zbody, 0)
    for j in range(RPT // CHUNK):
        pltpu.sync_copy(rows_v.at[0],
                        agg_s.at[pl.ds(s * RPT + j * CHUNK, CHUNK)])
    plsc.subcore_barrier()

    def gather(q, buf):
        return pltpu.async_copy(hwp_hbm.at[igrp.at[q, 0]], rows_v.at[buf],
                                gsem)

    def scat(q, buf):
        return pltpu.async_copy(rows_v.at[buf], agg_s.at[igrp.at[q, 1]],
                                ssem[q % 2], add=True)

    def group(g, carry):
        pltpu.sync_copy(sd_hbm.at[c, s, g], igrp)
        gather(0, 0).wait()
        scd = [None] * G
        for q in range(1, G):
            scd[q - 1] = scat(q - 1, (q - 1) % 2)
            if q >= 2:
                scd[q - 2].wait()
            gather(q, q % 2).wait()
        scd[G - 1] = scat(G - 1, (G - 1) % 2)
        scd[G - 2].wait()
        scd[G - 1].wait()
        return carry

    lax.fori_loop(0, NGRP, group, 0)
    plsc.subcore_barrier()

    pltpu.sync_copy(agg_s.at[pl.ds(s * RPT, RPT)],
                    out_hbm.at[pl.ds(c * N + s * RPT, RPT)])


def kernel(inp_h, edge_index_e0, edge_index_e1, W_e0, W_e1):
    src = jnp.stack([edge_index_e0[0], edge_index_e1[0] + N])
    dst = jnp.stack([edge_index_e0[1], edge_index_e1[1]])
    sd = jnp.stack([src.reshape(2, NT, NGRP, G, CHUNK),
                    dst.reshape(2, NT, NGRP, G, CHUNK)],
                   axis=4)
    wp = jnp.zeros((2, D, DP), jnp.float32)
    wp = wp.at[:, :, :D].set(jnp.stack([W_e0, W_e1]))

    hwp = pl.pallas_call(
        _mm_body,
        grid=(2, N // BM),
        in_specs=[
            pl.BlockSpec((BM, D), lambda r, i: (i, 0)),
            pl.BlockSpec((1, D, DP), lambda r, i: (r, 0, 0)),
        ],
        out_specs=pl.BlockSpec((BM, DP), lambda r, i: (r * (N // BM) + i, 0)),
        out_shape=jax.ShapeDtypeStruct((2 * N, DP), jnp.float32),
    )(inp_h, wp)

    agg = _sc_aggregate(hwp, sd)

    out = pl.pallas_call(
        _fin_body,
        grid=(N // BM,),
        in_specs=[
            pl.BlockSpec((BM, DP), lambda i: (i, 0)),
            pl.BlockSpec((BM, DP), lambda i: (i + N // BM, 0)),
        ],
        out_specs=pl.BlockSpec((BM, D), lambda i: (i, 0)),
        out_shape=jax.ShapeDtypeStruct((N, D), jnp.float32),
    )(agg, agg)
    return out


# speedup vs baseline: 2.3122x; 1.0419x over previous
import functools

import jax
import jax.numpy as jnp
from jax import lax
from jax.experimental import pallas as pl
from jax.experimental.pallas import tpu as pltpu
from jax.experimental.pallas import tpu_sc as plsc

N = 10000
E = 160000
D = 128
DP = 144
NT = 16
CHUNK = 125
G = 8
NGRP = (E // NT) // (G * CHUNK)
RPT = N // NT
BM = 1000


def _mm_body(h_ref, w_ref, o_ref):
    acc = jnp.dot(h_ref[...], w_ref[0], preferred_element_type=jnp.float32)
    col = lax.broadcasted_iota(jnp.int32, (BM, DP), 1)
    o_ref[...] = acc + jnp.where(col == D, 1.0, 0.0)


def _fin_body(a0_ref, a1_ref, o_ref):
    x0 = a0_ref[...]
    x1 = a1_ref[...]
    d0 = jnp.maximum(x0[:, D:D + 1], 1.0)
    d1 = jnp.maximum(x1[:, D:D + 1], 1.0)
    o_ref[...] = jnp.maximum(x0[:, :D] / d0 + x1[:, :D] / d1, 0.0)


@functools.partial(
    pl.kernel,
    out_type=jax.ShapeDtypeStruct((2 * N, DP), jnp.float32),
    mesh=plsc.VectorSubcoreMesh(core_axis_name="c", subcore_axis_name="s"),
    scratch_types=[
        pltpu.VMEM((2, G, 2, CHUNK), jnp.int32),
        pltpu.VMEM((2, CHUNK, DP), jnp.float32),
        pltpu.VMEM_SHARED((N, DP), jnp.float32),
        pltpu.SemaphoreType.DMA,
        pltpu.SemaphoreType.DMA,
        pltpu.SemaphoreType.DMA,
        pltpu.SemaphoreType.DMA,
        pltpu.SemaphoreType.DMA,
    ],
    compiler_params=pltpu.CompilerParams(use_tc_tiling_on_sc=False),
)
def _sc_aggregate(hwp_hbm, sd_hbm, out_hbm,
                  igrp, rows_v, agg_s, gsem, ss0, ss1, is0, is1):
    c = lax.axis_index("c")
    s = lax.axis_index("s")
    ssem = (ss0, ss1)
    isem = (is0, is1)

    zrow = jnp.zeros((16,), jnp.float32)

    def zbody(i, carry):
        for k in range(DP // 16):
            rows_v[0, i, pl.ds(k * 16, 16)] = zrow
        return carry

    lax.fori_loop(0, CHUNK, zbody, 0)
    for j in range(RPT // CHUNK):
        pltpu.sync_copy(rows_v.at[0],
                        agg_s.at[pl.ds(s * RPT + j * CHUNK, CHUNK)])
    plsc.subcore_barrier()

    def gather(p, q, buf):
        return pltpu.async_copy(hwp_hbm.at[igrp.at[p, q, 0]], rows_v.at[buf],
                                gsem)

    def scat(p, q, buf):
        return pltpu.async_copy(rows_v.at[buf], agg_s.at[igrp.at[p, q, 1]],
                                ssem[q % 2], add=True)

    def half(g, p):
        pltpu.make_async_copy(sd_hbm.at[c, s, 0], igrp.at[p],
                              isem[p]).wait()
        gather(p, 0, 0).wait()
        scd = [None] * G
        for q in range(1, G):
            scd[q - 1] = scat(p, q - 1, (q - 1) % 2)
            if q >= 2:
                scd[q - 2].wait()
            gather(p, q, q % 2).wait()
        scd[G - 1] = scat(p, G - 1, (G - 1) % 2)
        scd[G - 2].wait()
        scd[G - 1].wait()
        pltpu.async_copy(sd_hbm.at[c, s, jnp.minimum(g + 2, NGRP - 1)],
                         igrp.at[p], isem[p])

    pltpu.async_copy(sd_hbm.at[c, s, 0], igrp.at[0], isem[0])
    pltpu.async_copy(sd_hbm.at[c, s, 1], igrp.at[1], isem[1])

    def body(m, carry):
        half(2 * m, 0)
        half(2 * m + 1, 1)
        return carry

    lax.fori_loop(0, NGRP // 2, body, 0)
    for p in range(2):
        pltpu.make_async_copy(sd_hbm.at[c, s, 0], igrp.at[p],
                              isem[p]).wait()
    plsc.subcore_barrier()

    pltpu.sync_copy(agg_s.at[pl.ds(s * RPT, RPT)],
                    out_hbm.at[pl.ds(c * N + s * RPT, RPT)])


def kernel(inp_h, edge_index_e0, edge_index_e1, W_e0, W_e1):
    src = jnp.stack([edge_index_e0[0], edge_index_e1[0] + N])
    dst = jnp.stack([edge_index_e0[1], edge_index_e1[1]])
    sd = jnp.stack([src.reshape(2, NT, NGRP, G, CHUNK),
                    dst.reshape(2, NT, NGRP, G, CHUNK)],
                   axis=4)
    wp = jnp.zeros((2, D, DP), jnp.float32)
    wp = wp.at[:, :, :D].set(jnp.stack([W_e0, W_e1]))

    hwp = pl.pallas_call(
        _mm_body,
        grid=(2, N // BM),
        in_specs=[
            pl.BlockSpec((BM, D), lambda r, i: (i, 0)),
            pl.BlockSpec((1, D, DP), lambda r, i: (r, 0, 0)),
        ],
        out_specs=pl.BlockSpec((BM, DP), lambda r, i: (r * (N // BM) + i, 0)),
        out_shape=jax.ShapeDtypeStruct((2 * N, DP), jnp.float32),
    )(inp_h, wp)

    agg = _sc_aggregate(hwp, sd)

    out = pl.pallas_call(
        _fin_body,
        grid=(N // BM,),
        in_specs=[
            pl.BlockSpec((BM, DP), lambda i: (i, 0)),
            pl.BlockSpec((BM, DP), lambda i: (i + N // BM, 0)),
        ],
        out_specs=pl.BlockSpec((BM, D), lambda i: (i, 0)),
        out_shape=jax.ShapeDtypeStruct((N, D), jnp.float32),
    )(agg, agg)
    return out


# speedup vs baseline: 2.7554x; 1.1917x over previous
import functools

import jax
import jax.numpy as jnp
from jax import lax
from jax.experimental import pallas as pl
from jax.experimental.pallas import tpu as pltpu
from jax.experimental.pallas import tpu_sc as plsc

N = 10000
E = 160000
D = 128
NT = 16
CHUNK = 125
G = 8
NGRP = (E // NT) // (G * CHUNK)
RPT = N // NT
BM = 1000
LANE = 16


def _mm_body(h_ref, w_ref, o_ref):
    o_ref[...] = jnp.dot(h_ref[...], w_ref[0],
                         preferred_element_type=jnp.float32)


def _fin_body(a0_ref, a1_ref, o_ref):
    o_ref[...] = jnp.maximum(a0_ref[...] + a1_ref[...], 0.0)


@functools.partial(
    pl.kernel,
    out_type=jax.ShapeDtypeStruct((2 * N, D), jnp.float32),
    mesh=plsc.VectorSubcoreMesh(core_axis_name="c", subcore_axis_name="s"),
    scratch_types=[
        pltpu.VMEM((2, G, 2, CHUNK), jnp.int32),
        pltpu.VMEM((2, 128, D), jnp.float32),
        pltpu.VMEM((N // LANE, LANE), jnp.float32),
        pltpu.VMEM((CHUNK,), jnp.int32),
        pltpu.VMEM((8, LANE), jnp.float32),
        pltpu.VMEM_SHARED((N, D), jnp.float32),
        pltpu.VMEM_SHARED((N // LANE, LANE), jnp.float32),
        pltpu.SemaphoreType.DMA,
        pltpu.SemaphoreType.DMA,
        pltpu.SemaphoreType.DMA,
        pltpu.SemaphoreType.DMA,
        pltpu.SemaphoreType.DMA,
    ],
    compiler_params=pltpu.CompilerParams(use_tc_tiling_on_sc=False,
                                         needs_layout_passes=False),
)
def _sc_aggregate(hw_hbm, sd_hbm, out_hbm,
                  igrp, rows_v, degp, iota_v, degc, agg_s, degs,
                  gsem, ss0, ss1, is0, is1):
    c = lax.axis_index("c")
    s = lax.axis_index("s")
    ssem = (ss0, ss1)
    isem = (is0, is1)
    zrow = jnp.zeros((LANE,), jnp.float32)
    ones = jnp.ones((LANE,), jnp.float32)
    lanes = lax.iota(jnp.int32, LANE)

    def zdeg(i, carry):
        degp[i] = zrow
        return carry

    lax.fori_loop(0, N // LANE, zdeg, 0)

    def zbody(i, carry):
        for k in range(D // LANE):
            rows_v[0, i, pl.ds(k * LANE, LANE)] = zrow
        return carry

    lax.fori_loop(0, CHUNK, zbody, 0)
    for j in range(RPT // CHUNK):
        pltpu.sync_copy(rows_v.at[0, pl.ds(0, CHUNK)],
                        agg_s.at[pl.ds(s * RPT + j * CHUNK, CHUNK)])

    @pl.when(s == 0)
    def _():
        for j in range(5):
            pltpu.sync_copy(degp.at[pl.ds(j * CHUNK, CHUNK)],
                            degs.at[pl.ds(j * CHUNK, CHUNK)])

    plsc.subcore_barrier()

    def gather(p, q, buf):
        return pltpu.async_copy(hw_hbm.at[igrp.at[p, q, 0]],
                                rows_v.at[buf, pl.ds(0, CHUNK)], gsem)

    def scat(p, q, buf):
        return pltpu.async_copy(rows_v.at[buf, pl.ds(0, CHUNK)],
                                agg_s.at[igrp.at[p, q, 1]],
                                ssem[q % 2], add=True)

    def count(p, q):
        for k in range(CHUNK // LANE):
            d = igrp[p, q, 1, pl.ds(k * LANE, LANE)]
            plsc.addupdate_scatter(degp, [d >> 4, d & 15], ones)
        d = igrp[p, q, 1, pl.ds(CHUNK - LANE, LANE)]
        plsc.addupdate_scatter(degp, [d >> 4, d & 15], ones,
                               mask=lanes >= (LANE - CHUNK % LANE))

    def half(g, p):
        pltpu.make_async_copy(sd_hbm.at[c, s, 0], igrp.at[p],
                              isem[p]).wait()
        gd = gather(p, 0, 0)
        for q in range(G):
            count(p, q)
        gd.wait()
        scd = [None] * G
        for q in range(1, G):
            scd[q - 1] = scat(p, q - 1, (q - 1) % 2)
            if q >= 2:
                scd[q - 2].wait()
            gather(p, q, q % 2).wait()
        scd[G - 1] = scat(p, G - 1, (G - 1) % 2)
        scd[G - 2].wait()
        scd[G - 1].wait()
        pltpu.async_copy(sd_hbm.at[c, s, jnp.minimum(g + 2, NGRP - 1)],
                         igrp.at[p], isem[p])

    pltpu.async_copy(sd_hbm.at[c, s, 0], igrp.at[0], isem[0])
    pltpu.async_copy(sd_hbm.at[c, s, 1], igrp.at[1], isem[1])

    def body(m, carry):
        half(2 * m, 0)
        half(2 * m + 1, 1)
        return carry

    lax.fori_loop(0, NGRP // 2, body, 0)
    for p in range(2):
        pltpu.make_async_copy(sd_hbm.at[c, s, 0], igrp.at[p],
                              isem[p]).wait()

    for j in range(5):
        base = j * CHUNK
        for k in range(CHUNK // LANE):
            iota_v[pl.ds(k * LANE, LANE)] = lanes + (base + k * LANE)
        iota_v[pl.ds(CHUNK - LANE, LANE)] = lanes + (base + CHUNK - LANE)
        pltpu.sync_copy(degp.at[pl.ds(base, CHUNK)],
                        degs.at[iota_v], add=True)
    plsc.subcore_barrier()

    def norm_chunk(base, nrows):
        pltpu.sync_copy(agg_s.at[pl.ds(base, nrows)],
                        rows_v.at[0, pl.ds(0, nrows)])
        pltpu.sync_copy(degs.at[pl.ds(base // LANE, nrows // LANE)],
                        degc.at[pl.ds(0, nrows // LANE)])

        def nrow(r, carry):
            d16 = degc[r >> 4]
            inv16 = 1.0 / jnp.maximum(d16, 1.0)
            inv = lax.gather(
                inv16, jnp.full((LANE, 1), r & 15, jnp.int32),
                lax.GatherDimensionNumbers(offset_dims=(),
                                           collapsed_slice_dims=(0,),
                                           start_index_map=(0,)),
                (1,), mode=lax.GatherScatterMode.PROMISE_IN_BOUNDS)
            for k in range(D // LANE):
                rows_v[0, r, pl.ds(k * LANE, LANE)] = (
                    rows_v[0, r, pl.ds(k * LANE, LANE)] * inv)
            return carry

        lax.fori_loop(0, nrows, nrow, 0)
        pltpu.sync_copy(rows_v.at[0, pl.ds(0, nrows)],
                        out_hbm.at[pl.ds(c * N + base, nrows)])

    @pl.when(s < 15)
    def _():
        for t in range(5):
            norm_chunk(s * 640 + t * 128, 128)

    @pl.when(s == 15)
    def _():
        for t in range(3):
            norm_chunk(9600 + t * 128, 128)
        norm_chunk(9984, 16)


def kernel(inp_h, edge_index_e0, edge_index_e1, W_e0, W_e1):
    src = jnp.stack([edge_index_e0[0], edge_index_e1[0] + N])
    dst = jnp.stack([edge_index_e0[1], edge_index_e1[1]])
    sd = jnp.stack([src.reshape(2, NT, NGRP, G, CHUNK),
                    dst.reshape(2, NT, NGRP, G, CHUNK)],
                   axis=4)
    w = jnp.stack([W_e0, W_e1])

    hw = pl.pallas_call(
        _mm_body,
        grid=(2, N // BM),
        in_specs=[
            pl.BlockSpec((BM, D), lambda r, i: (i, 0)),
            pl.BlockSpec((1, D, D), lambda r, i: (r, 0, 0)),
        ],
        out_specs=pl.BlockSpec((BM, D), lambda r, i: (r * (N // BM) + i, 0)),
        out_shape=jax.ShapeDtypeStruct((2 * N, D), jnp.float32),
    )(inp_h, w)

    agg = _sc_aggregate(hw, sd)

    out = pl.pallas_call(
        _fin_body,
        grid=(N // BM,),
        in_specs=[
            pl.BlockSpec((BM, D), lambda i: (i, 0)),
            pl.BlockSpec((BM, D), lambda i: (i + N // BM, 0)),
        ],
        out_specs=pl.BlockSpec((BM, D), lambda i: (i, 0)),
        out_shape=jax.ShapeDtypeStruct((N, D), jnp.float32),
    )(agg, agg)
    return out


# speedup vs baseline: 2.7642x; 1.0032x over previous
import functools

import jax
import jax.numpy as jnp
from jax import lax
from jax.experimental import pallas as pl
from jax.experimental.pallas import tpu as pltpu
from jax.experimental.pallas import tpu_sc as plsc

N = 10000
E = 160000
D = 128
NT = 16
CHUNK = 125
G = 10
NGRP = (E // NT) // (G * CHUNK)
RPT = N // NT
BM = 1000
LANE = 16


def _mm_body(h_ref, w_ref, o_ref):
    o_ref[...] = jnp.dot(h_ref[...], w_ref[0],
                         preferred_element_type=jnp.float32)


def _fin_body(a0_ref, a1_ref, o_ref):
    o_ref[...] = jnp.maximum(a0_ref[...] + a1_ref[...], 0.0)


@functools.partial(
    pl.kernel,
    out_type=jax.ShapeDtypeStruct((2 * N, D), jnp.float32),
    mesh=plsc.VectorSubcoreMesh(core_axis_name="c", subcore_axis_name="s"),
    scratch_types=[
        pltpu.VMEM((2, G, 2, CHUNK), jnp.int32),
        pltpu.VMEM((2, 128, D), jnp.float32),
        pltpu.VMEM((N // LANE, LANE), jnp.float32),
        pltpu.VMEM((CHUNK,), jnp.int32),
        pltpu.VMEM((8, LANE), jnp.float32),
        pltpu.VMEM_SHARED((N, D), jnp.float32),
        pltpu.VMEM_SHARED((N // LANE, LANE), jnp.float32),
        pltpu.SemaphoreType.DMA,
        pltpu.SemaphoreType.DMA,
        pltpu.SemaphoreType.DMA,
        pltpu.SemaphoreType.DMA,
        pltpu.SemaphoreType.DMA,
    ],
    compiler_params=pltpu.CompilerParams(use_tc_tiling_on_sc=False,
                                         needs_layout_passes=False),
)
def _sc_aggregate(hw_hbm, sd_hbm, out_hbm,
                  igrp, rows_v, degp, iota_v, degc, agg_s, degs,
                  gsem, ss0, ss1, is0, is1):
    c = lax.axis_index("c")
    s = lax.axis_index("s")
    ssem = (ss0, ss1)
    isem = (is0, is1)
    zrow = jnp.zeros((LANE,), jnp.float32)
    ones = jnp.ones((LANE,), jnp.float32)
    lanes = lax.iota(jnp.int32, LANE)

    def zdeg(i, carry):
        degp[i] = zrow
        return carry

    lax.fori_loop(0, N // LANE, zdeg, 0)

    def zbody(i, carry):
        for k in range(D // LANE):
            rows_v[0, i, pl.ds(k * LANE, LANE)] = zrow
        return carry

    lax.fori_loop(0, CHUNK, zbody, 0)
    for j in range(RPT // CHUNK):
        pltpu.sync_copy(rows_v.at[0, pl.ds(0, CHUNK)],
                        agg_s.at[pl.ds(s * RPT + j * CHUNK, CHUNK)])

    @pl.when(s == 0)
    def _():
        for j in range(5):
            pltpu.sync_copy(degp.at[pl.ds(j * CHUNK, CHUNK)],
                            degs.at[pl.ds(j * CHUNK, CHUNK)])

    plsc.subcore_barrier()

    def gather(p, q, buf):
        return pltpu.async_copy(hw_hbm.at[igrp.at[p, q, 0]],
                                rows_v.at[buf, pl.ds(0, CHUNK)], gsem)

    def scat(p, q, buf):
        return pltpu.async_copy(rows_v.at[buf, pl.ds(0, CHUNK)],
                                agg_s.at[igrp.at[p, q, 1]],
                                ssem[q % 2], add=True)

    def count(p, q):
        for k in range(CHUNK // LANE):
            d = igrp[p, q, 1, pl.ds(k * LANE, LANE)]
            plsc.addupdate_scatter(degp, [d >> 4, d & 15], ones)
        d = igrp[p, q, 1, pl.ds(CHUNK - LANE, LANE)]
        plsc.addupdate_scatter(degp, [d >> 4, d & 15], ones,
                               mask=lanes >= (LANE - CHUNK % LANE))

    def half(g, p):
        pltpu.make_async_copy(sd_hbm.at[c, s, 0], igrp.at[p],
                              isem[p]).wait()
        gd = gather(p, 0, 0)
        for q in range(G):
            count(p, q)
        gd.wait()
        scd = [None] * G
        for q in range(1, G):
            scd[q - 1] = scat(p, q - 1, (q - 1) % 2)
            if q >= 2:
                scd[q - 2].wait()
            gather(p, q, q % 2).wait()
        scd[G - 1] = scat(p, G - 1, (G - 1) % 2)
        scd[G - 2].wait()
        scd[G - 1].wait()
        pltpu.async_copy(sd_hbm.at[c, s, jnp.minimum(g + 2, NGRP - 1)],
                         igrp.at[p], isem[p])

    pltpu.async_copy(sd_hbm.at[c, s, 0], igrp.at[0], isem[0])
    pltpu.async_copy(sd_hbm.at[c, s, 1], igrp.at[1], isem[1])

    def body(m, carry):
        half(2 * m, 0)
        half(2 * m + 1, 1)
        return carry

    lax.fori_loop(0, NGRP // 2, body, 0)
    for p in range(2):
        pltpu.make_async_copy(sd_hbm.at[c, s, 0], igrp.at[p],
                              isem[p]).wait()

    for j in range(5):
        base = j * CHUNK
        for k in range(CHUNK // LANE):
            iota_v[pl.ds(k * LANE, LANE)] = lanes + (base + k * LANE)
        iota_v[pl.ds(CHUNK - LANE, LANE)] = lanes + (base + CHUNK - LANE)
        pltpu.sync_copy(degp.at[pl.ds(base, CHUNK)],
                        degs.at[iota_v], add=True)
    plsc.subcore_barrier()

    def norm_chunk(base, nrows):
        pltpu.sync_copy(agg_s.at[pl.ds(base, nrows)],
                        rows_v.at[0, pl.ds(0, nrows)])
        pltpu.sync_copy(degs.at[pl.ds(base // LANE, nrows // LANE)],
                        degc.at[pl.ds(0, nrows // LANE)])

        def nrow(r, carry):
            d16 = degc[r >> 4]
            inv16 = 1.0 / jnp.maximum(d16, 1.0)
            inv = lax.gather(
                inv16, jnp.full((LANE, 1), r & 15, jnp.int32),
                lax.GatherDimensionNumbers(offset_dims=(),
                                           collapsed_slice_dims=(0,),
                                           start_index_map=(0,)),
                (1,), mode=lax.GatherScatterMode.PROMISE_IN_BOUNDS)
            for k in range(D // LANE):
                rows_v[0, r, pl.ds(k * LANE, LANE)] = (
                    rows_v[0, r, pl.ds(k * LANE, LANE)] * inv)
            return carry

        lax.fori_loop(0, nrows, nrow, 0)
        pltpu.sync_copy(rows_v.at[0, pl.ds(0, nrows)],
                        out_hbm.at[pl.ds(c * N + base, nrows)])

    @pl.when(s < 15)
    def _():
        for t in range(5):
            norm_chunk(s * 640 + t * 128, 128)

    @pl.when(s == 15)
    def _():
        for t in range(3):
            norm_chunk(9600 + t * 128, 128)
        norm_chunk(9984, 16)


def kernel(inp_h, edge_index_e0, edge_index_e1, W_e0, W_e1):
    src = jnp.stack([edge_index_e0[0], edge_index_e1[0] + N])
    dst = jnp.stack([edge_index_e0[1], edge_index_e1[1]])
    sd = jnp.stack([src.reshape(2, NT, NGRP, G, CHUNK),
                    dst.reshape(2, NT, NGRP, G, CHUNK)],
                   axis=4)
    w = jnp.stack([W_e0, W_e1])

    hw = pl.pallas_call(
        _mm_body,
        grid=(2, N // BM),
        in_specs=[
            pl.BlockSpec((BM, D), lambda r, i: (i, 0)),
            pl.BlockSpec((1, D, D), lambda r, i: (r, 0, 0)),
        ],
        out_specs=pl.BlockSpec((BM, D), lambda r, i: (r * (N // BM) + i, 0)),
        out_shape=jax.ShapeDtypeStruct((2 * N, D), jnp.float32),
    )(inp_h, w)

    agg = _sc_aggregate(hw, sd)

    out = pl.pallas_call(
        _fin_body,
        grid=(N // BM,),
        in_specs=[
            pl.BlockSpec((BM, D), lambda i: (i, 0)),
            pl.BlockSpec((BM, D), lambda i: (i + N // BM, 0)),
        ],
        out_specs=pl.BlockSpec((BM, D), lambda i: (i, 0)),
        out_shape=jax.ShapeDtypeStruct((N, D), jnp.float32),
    )(agg, agg)
    return out
